# baseline (speedup 1.0000x reference)
import functools

import jax
import jax.numpy as jnp
from jax.experimental import pallas as pl
from jax.experimental.pallas import tpu as pltpu

NUM_QUERIES = 300
NUM_CLASSES = 10
TOPK = 100

_FLAT = NUM_QUERIES * NUM_CLASSES
_PAD_ROWS = 24
_M = 16 * 96 * 96
_BM = 1536
_STEPS = _M // _BM
_D = 16


def _fused_kernel(probs_ref, masks_ref, boxes_ref,
                  binout_ref, misc_ref, boxout_ref,
                  onehot_s, vals_s, inbuf, sems):
    i = pl.program_id(0)

    def in_copy(step, slot):
        return pltpu.make_async_copy(
            masks_ref.at[pl.ds(step * _BM, _BM)],
            inbuf.at[slot],
            sems.at[slot])

    @pl.when(i == 0)
    def _():
        for s in range(_D):
            in_copy(s, s).start()

        x = probs_ref[...]
        r24 = jax.lax.broadcasted_iota(jnp.int32, (_PAD_ROWS, 128), 0)
        c24 = jax.lax.broadcasted_iota(jnp.int32, (_PAD_ROWS, 128), 1)
        flat = r24 * 128 + c24
        r8 = jax.lax.broadcasted_iota(jnp.int32, (8, 128), 0)
        c8 = jax.lax.broadcasted_iota(jnp.int32, (8, 128), 1)

        def body(k, carry):
            x, vacc, iacc = carry
            m = jnp.max(x)
            chosen = jnp.min(jnp.where(x == m, flat, jnp.int32(1 << 30)))
            x = jnp.where(flat == chosen, jnp.float32(-1.0), x)
            sel = (r8 == 0) & (c8 == k)
            vacc = jnp.where(sel, m, vacc)
            iacc = jnp.where(sel, chosen // NUM_CLASSES, iacc)
            return x, vacc, iacc

        _, vacc, iacc = jax.lax.fori_loop(
            0, TOPK, body,
            (x, jnp.zeros((8, 128), jnp.float32),
             jnp.zeros((8, 128), jnp.int32)),
        )
        vals_s[...] = vacc
        qrow = iacc[0:1, :]
        riota = jax.lax.broadcasted_iota(jnp.int32, (NUM_QUERIES, 128), 0)
        w0 = jnp.where(riota == qrow, jnp.float32(1.0), jnp.float32(0.0))
        onehot_s[...] = w0

        misc_ref[...] = jnp.zeros_like(misc_ref)
        bg = jax.lax.dot_general(
            boxes_ref[...], w0, (((1,), (0,)), ((), ())),
            precision=jax.lax.Precision.HIGHEST,
            preferred_element_type=jnp.float32)
        boxout_ref[...] = bg[:, :TOPK]

    slot = jax.lax.rem(i, _D)
    in_copy(i, slot).wait()
    x = inbuf[slot]
    w = onehot_s[...]
    sgn = jnp.where(x > 0, jnp.float32(1.0), jnp.float32(0.0))
    binf = jax.lax.dot_general(
        sgn, w, (((1,), (0,)), ((), ())),
        preferred_element_type=jnp.float32)
    binout_ref[...] = binf[:, :TOPK]
    g = jax.lax.dot_general(
        x, w, (((1,), (0,)), ((), ())),
        preferred_element_type=jnp.float32)
    th = jnp.tanh(g * 0.5)
    tpart = jnp.sum(th * binf, axis=0, keepdims=True)
    bpart = jnp.sum(binf, axis=0, keepdims=True)

    misc_ref[0:1, :] = misc_ref[0:1, :] + tpart
    misc_ref[1:2, :] = misc_ref[1:2, :] + bpart

    @pl.when(i + _D < _STEPS)
    def _():
        in_copy(i + _D, slot).start()

    @pl.when(i == _STEPS - 1)
    def _():
        t = misc_ref[0:1, :]
        b = misc_ref[1:2, :]
        conf = (0.5 * b + 0.5 * t) / (b + 1e-6)
        misc_ref[2:3, :] = vals_s[0:1, :] * conf


def kernel(predicted_labels, predicted_masks, predicted_boxes):
    probs = jax.nn.sigmoid(predicted_labels)
    flat = probs.reshape(-1)
    padded = jnp.concatenate(
        [flat, jnp.full((_PAD_ROWS * 128 - _FLAT,), -1.0, jnp.float32)]
    ).reshape(_PAD_ROWS, 128)

    masks2 = predicted_masks.transpose(1, 2, 3, 0).reshape(_M, NUM_QUERIES)
    boxes_t = predicted_boxes.transpose(1, 0)

    binout, misc, boxout = pl.pallas_call(
        _fused_kernel,
        grid=(_STEPS,),
        in_specs=[
            pl.BlockSpec((_PAD_ROWS, 128), lambda i: (0, 0)),
            pl.BlockSpec(memory_space=pltpu.MemorySpace.HBM),
            pl.BlockSpec((6, NUM_QUERIES), lambda i: (0, 0)),
        ],
        out_specs=[
            pl.BlockSpec((_BM, TOPK), lambda i: (i, 0)),
            pl.BlockSpec((8, 128), lambda i: (0, 0)),
            pl.BlockSpec((6, TOPK), lambda i: (0, 0)),
        ],
        out_shape=[
            jax.ShapeDtypeStruct((_M, TOPK), jnp.float32),
            jax.ShapeDtypeStruct((8, 128), jnp.float32),
            jax.ShapeDtypeStruct((6, TOPK), jnp.float32),
        ],
        scratch_shapes=[
            pltpu.VMEM((NUM_QUERIES, 128), jnp.float32),
            pltpu.VMEM((8, 128), jnp.float32),
            pltpu.VMEM((_D, _BM, NUM_QUERIES), jnp.float32),
            pltpu.SemaphoreType.DMA((_D,)),
        ],
    )(padded, masks2, boxes_t)

    labels_out = misc[2, :TOPK]
    boxes_sel = boxout.transpose(1, 0)
    masks_bin = binout.reshape(16, 96, 96, TOPK).transpose(3, 0, 1, 2)
    return (labels_out, boxes_sel, masks_bin)
